# baseline (speedup 1.0000x reference)
import functools
import jax
import jax.numpy as jnp
from jax import lax
from jax.experimental import pallas as pl
from jax.experimental.pallas import tpu as pltpu
from jax.experimental.pallas import tpu_sc as plsc


S_BLK = 2048


def _add_pe_kernel(x_ref, pe_ref, o_ref):
    o_ref[...] = x_ref[...] + pe_ref[...]


def _tc_kernel(x, pos_table):
    b, s, d = x.shape
    grid = (s // S_BLK, b)
    return pl.pallas_call(
        _add_pe_kernel,
        grid=grid,
        in_specs=[
            pl.BlockSpec((1, S_BLK, d), lambda i, j: (j, i, 0)),
            pl.BlockSpec((S_BLK, d), lambda i, j: (i, 0)),
        ],
        out_specs=pl.BlockSpec((1, S_BLK, d), lambda i, j: (j, i, 0)),
        out_shape=jax.ShapeDtypeStruct((b, s, d), x.dtype),
    )(x, pos_table)


NW = 32
CH = 16


def _sc_add_pe(x, pos_table):
    b, s, d = x.shape
    rows = b * s
    rw = rows // NW
    nch = rw // CH

    x2 = x.reshape(rows, d)
    mesh = plsc.VectorSubcoreMesh(core_axis_name="c", subcore_axis_name="s")

    @functools.partial(
        pl.kernel,
        out_type=jax.ShapeDtypeStruct((rows, d), jnp.float32),
        mesh=mesh,
        scratch_types=[
            pltpu.VMEM((2, CH, d), jnp.float32),
            pltpu.VMEM((2, CH, d), jnp.float32),
            pltpu.VMEM((2, CH, d), jnp.float32),
            pltpu.SemaphoreType.DMA,
            pltpu.SemaphoreType.DMA,
            pltpu.SemaphoreType.DMA,
            pltpu.SemaphoreType.DMA,
            pltpu.SemaphoreType.DMA,
            pltpu.SemaphoreType.DMA,
        ],
    )
    def k(x_hbm, tab_hbm, out_hbm, xbuf, tbuf, obuf,
          lx0, lx1, lt0, lt1, st0, st1):
        wid = lax.axis_index("s") * 2 + lax.axis_index("c")
        row0 = wid * rw
        pos0 = row0 % s
        lxs = (lx0, lx1)
        lts = (lt0, lt1)
        sts = (st0, st1)

        def issue_loads(c, slot):
            pltpu.async_copy(x_hbm.at[pl.ds(row0 + c * CH, CH)],
                             xbuf.at[slot], lxs[slot])
            pltpu.async_copy(tab_hbm.at[pl.ds(pos0 + c * CH, CH)],
                             tbuf.at[slot], lts[slot])

        issue_loads(0, 0)

        @pl.loop(0, nch // 2)
        def _pair(i):
            for half in (0, 1):
                c = i * 2 + half
                slot = half

                @pl.when(c + 1 < nch)
                def _():
                    issue_loads(c + 1, 1 - slot)

                @pl.when(c >= 2)
                def _():
                    pltpu.make_async_copy(
                        obuf.at[slot],
                        out_hbm.at[pl.ds(row0 + (c - 2) * CH, CH)],
                        sts[slot],
                    ).wait()

                pltpu.make_async_copy(
                    x_hbm.at[pl.ds(row0 + c * CH, CH)],
                    xbuf.at[slot], lxs[slot]).wait()
                pltpu.make_async_copy(
                    tab_hbm.at[pl.ds(pos0 + c * CH, CH)],
                    tbuf.at[slot], lts[slot]).wait()

                @plsc.parallel_loop(0, CH, unroll=2)
                def _row(r):
                    for j in range(d // 16):
                        obuf[slot, r, pl.ds(j * 16, 16)] = (
                            xbuf[slot, r, pl.ds(j * 16, 16)]
                            + tbuf[slot, r, pl.ds(j * 16, 16)]
                        )

                pltpu.async_copy(
                    obuf.at[slot],
                    out_hbm.at[pl.ds(row0 + c * CH, CH)],
                    sts[slot])

        for c in (nch - 2, nch - 1):
            pltpu.make_async_copy(
                obuf.at[c % 2],
                out_hbm.at[pl.ds(row0 + c * CH, CH)],
                sts[c % 2],
            ).wait()

    return k(x2, pos_table).reshape(b, s, d)


def kernel(x, pos_table):
    return _sc_add_pe(x, pos_table)


# speedup vs baseline: 1.8013x; 1.8013x over previous
import functools
import jax
import jax.numpy as jnp
from jax import lax
from jax.experimental import pallas as pl
from jax.experimental.pallas import tpu as pltpu
from jax.experimental.pallas import tpu_sc as plsc


S_BLK = 2048


def _add_pe_kernel(x_ref, pe_ref, o_ref):
    o_ref[...] = x_ref[...] + pe_ref[...]


def _tc_kernel(x, pos_table):
    b, s, d = x.shape
    grid = (s // S_BLK, b)
    return pl.pallas_call(
        _add_pe_kernel,
        grid=grid,
        in_specs=[
            pl.BlockSpec((1, S_BLK, d), lambda i, j: (j, i, 0)),
            pl.BlockSpec((S_BLK, d), lambda i, j: (i, 0)),
        ],
        out_specs=pl.BlockSpec((1, S_BLK, d), lambda i, j: (j, i, 0)),
        out_shape=jax.ShapeDtypeStruct((b, s, d), x.dtype),
    )(x, pos_table)


NW = 32
CH = 16


def _sc_add_pe(x, pos_table):
    b, s, d = x.shape
    rows = b * s
    rw = rows // NW
    nch = rw // CH

    x2 = x.reshape(rows, d)
    mesh = plsc.VectorSubcoreMesh(core_axis_name="c", subcore_axis_name="s")

    nb = b
    sw = s // NW
    nsc = sw // CH

    @functools.partial(
        pl.kernel,
        out_type=jax.ShapeDtypeStruct((rows, d), jnp.float32),
        mesh=mesh,
        scratch_types=[
            pltpu.VMEM((2, CH, d), jnp.float32),
            pltpu.VMEM((2, CH, d), jnp.float32),
            pltpu.VMEM((2, CH, d), jnp.float32),
            pltpu.SemaphoreType.DMA,
            pltpu.SemaphoreType.DMA,
            pltpu.SemaphoreType.DMA,
            pltpu.SemaphoreType.DMA,
            pltpu.SemaphoreType.DMA,
            pltpu.SemaphoreType.DMA,
        ],
    )
    def k(x_hbm, tab_hbm, out_hbm, xbuf, tbuf, obuf,
          lx0, lx1, lt0, lt1, st0, st1):
        wid = lax.axis_index("s") * 2 + lax.axis_index("c")
        s0 = wid * sw
        lxs = (lx0, lx1)
        lts = (lt0, lt1)
        sts = (st0, st1)

        def xrow(c):
            return (c % nb) * s + s0 + (c // nb) * CH

        def issue_x(c, slot):
            pltpu.async_copy(x_hbm.at[pl.ds(xrow(c), CH)],
                             xbuf.at[slot], lxs[slot])

        def issue_t(sc, tslot):
            pltpu.async_copy(tab_hbm.at[pl.ds(s0 + sc * CH, CH)],
                             tbuf.at[tslot], lts[tslot])

        issue_x(0, 0)
        issue_t(0, 0)

        @pl.loop(0, nsc // 2)
        def _scpair(i2):
            for scp in (0, 1):
                sc = i2 * 2 + scp
                tslot = scp
                for bb in range(nb):
                    c = sc * nb + bb
                    slot = bb % 2

                    @pl.when(c + 1 < nch)
                    def _():
                        issue_x(c + 1, 1 - slot)

                    if bb == nb - 1:
                        @pl.when(sc + 1 < nsc)
                        def _():
                            issue_t(sc + 1, 1 - tslot)

                    @pl.when(c >= 2)
                    def _():
                        pltpu.make_async_copy(
                            obuf.at[slot],
                            out_hbm.at[pl.ds(xrow(c - 2), CH)],
                            sts[slot],
                        ).wait()

                    pltpu.make_async_copy(
                        x_hbm.at[pl.ds(xrow(c), CH)],
                        xbuf.at[slot], lxs[slot]).wait()

                    if bb == 0:
                        pltpu.make_async_copy(
                            tab_hbm.at[pl.ds(s0 + sc * CH, CH)],
                            tbuf.at[tslot], lts[tslot]).wait()

                    nj = d // 16

                    @plsc.parallel_loop(0, CH * nj, unroll=4)
                    def _q(q):
                        r = q // nj
                        j = (q % nj) * 16
                        obuf[slot, r, pl.ds(j, 16)] = (
                            xbuf[slot, r, pl.ds(j, 16)]
                            + tbuf[tslot, r, pl.ds(j, 16)]
                        )

                    pltpu.async_copy(
                        obuf.at[slot],
                        out_hbm.at[pl.ds(xrow(c), CH)],
                        sts[slot])

        for c in (nch - 2, nch - 1):
            pltpu.make_async_copy(
                obuf.at[c % 2],
                out_hbm.at[pl.ds(xrow(c), CH)],
                sts[c % 2],
            ).wait()

    return k(x2, pos_table).reshape(b, s, d)


def kernel(x, pos_table):
    return _sc_add_pe(x, pos_table)


# speedup vs baseline: 1.8396x; 1.0213x over previous
import functools
import jax
import jax.numpy as jnp
from jax import lax
from jax.experimental import pallas as pl
from jax.experimental.pallas import tpu as pltpu
from jax.experimental.pallas import tpu_sc as plsc


S_BLK = 2048


def _add_pe_kernel(x_ref, pe_ref, o_ref):
    o_ref[...] = x_ref[...] + pe_ref[...]


def _tc_kernel(x, pos_table):
    b, s, d = x.shape
    grid = (s // S_BLK, b)
    return pl.pallas_call(
        _add_pe_kernel,
        grid=grid,
        in_specs=[
            pl.BlockSpec((1, S_BLK, d), lambda i, j: (j, i, 0)),
            pl.BlockSpec((S_BLK, d), lambda i, j: (i, 0)),
        ],
        out_specs=pl.BlockSpec((1, S_BLK, d), lambda i, j: (j, i, 0)),
        out_shape=jax.ShapeDtypeStruct((b, s, d), x.dtype),
    )(x, pos_table)


NW = 32
CH = 16


def _sc_add_pe(x, pos_table):
    b, s, d = x.shape
    rows = b * s
    rw = rows // NW
    nch = rw // CH

    x2 = x.reshape(rows, d)
    mesh = plsc.VectorSubcoreMesh(core_axis_name="c", subcore_axis_name="s")

    nb = b
    sw = s // NW
    nsc = sw // CH

    @functools.partial(
        pl.kernel,
        out_type=jax.ShapeDtypeStruct((rows, d), jnp.float32),
        mesh=mesh,
        scratch_types=[
            pltpu.VMEM((2, CH, d), jnp.float32),
            pltpu.VMEM((2, CH, d), jnp.float32),
            pltpu.VMEM((2, CH, d), jnp.float32),
            pltpu.SemaphoreType.DMA,
            pltpu.SemaphoreType.DMA,
            pltpu.SemaphoreType.DMA,
            pltpu.SemaphoreType.DMA,
            pltpu.SemaphoreType.DMA,
            pltpu.SemaphoreType.DMA,
        ],
    )
    def k(x_hbm, tab_hbm, out_hbm, xbuf, tbuf, obuf,
          lx0, lx1, lt0, lt1, st0, st1):
        wid = lax.axis_index("s") * 2 + lax.axis_index("c")
        s0 = wid * sw
        lxs = (lx0, lx1)
        lts = (lt0, lt1)
        sts = (st0, st1)

        def xrow(c):
            return (c % nb) * s + s0 + (c // nb) * CH

        def issue_x(c, slot):
            pltpu.async_copy(x_hbm.at[pl.ds(xrow(c), CH)],
                             xbuf.at[slot], lxs[slot])

        def issue_t(sc, tslot):
            pltpu.async_copy(tab_hbm.at[pl.ds(s0 + sc * CH, CH)],
                             tbuf.at[tslot], lts[tslot])

        issue_x(0, 0)
        issue_t(0, 0)

        @pl.loop(0, nsc // 2)
        def _scpair(i2):
            for scp in (0, 1):
                sc = i2 * 2 + scp
                tslot = scp
                for bb in range(nb):
                    c = sc * nb + bb
                    slot = bb % 2

                    @pl.when(c + 1 < nch)
                    def _():
                        issue_x(c + 1, 1 - slot)

                    if bb == nb - 1:
                        @pl.when(sc + 1 < nsc)
                        def _():
                            issue_t(sc + 1, 1 - tslot)

                    @pl.when(c >= 2)
                    def _():
                        pltpu.make_async_copy(
                            obuf.at[slot],
                            out_hbm.at[pl.ds(xrow(c - 2), CH)],
                            sts[slot],
                        ).wait()

                    pltpu.make_async_copy(
                        x_hbm.at[pl.ds(xrow(c), CH)],
                        xbuf.at[slot], lxs[slot]).wait()

                    if bb == 0:
                        pltpu.make_async_copy(
                            tab_hbm.at[pl.ds(s0 + sc * CH, CH)],
                            tbuf.at[tslot], lts[tslot]).wait()

                    nj = d // 16

                    @plsc.parallel_loop(0, CH * nj, unroll=8)
                    def _q(q):
                        r = q // nj
                        j = (q % nj) * 16
                        obuf[slot, r, pl.ds(j, 16)] = (
                            xbuf[slot, r, pl.ds(j, 16)]
                            + tbuf[tslot, r, pl.ds(j, 16)]
                        )

                    pltpu.async_copy(
                        obuf.at[slot],
                        out_hbm.at[pl.ds(xrow(c), CH)],
                        sts[slot])

        for c in (nch - 2, nch - 1):
            pltpu.make_async_copy(
                obuf.at[c % 2],
                out_hbm.at[pl.ds(xrow(c), CH)],
                sts[c % 2],
            ).wait()

    return k(x2, pos_table).reshape(b, s, d)


def kernel(x, pos_table):
    return _sc_add_pe(x, pos_table)


# speedup vs baseline: 1.8810x; 1.0225x over previous
import functools
import jax
import jax.numpy as jnp
from jax import lax
from jax.experimental import pallas as pl
from jax.experimental.pallas import tpu as pltpu
from jax.experimental.pallas import tpu_sc as plsc


S_BLK = 2048


def _add_pe_kernel(x_ref, pe_ref, o_ref):
    o_ref[...] = x_ref[...] + pe_ref[...]


def _tc_kernel(x, pos_table):
    b, s, d = x.shape
    grid = (s // S_BLK, b)
    return pl.pallas_call(
        _add_pe_kernel,
        grid=grid,
        in_specs=[
            pl.BlockSpec((1, S_BLK, d), lambda i, j: (j, i, 0)),
            pl.BlockSpec((S_BLK, d), lambda i, j: (i, 0)),
        ],
        out_specs=pl.BlockSpec((1, S_BLK, d), lambda i, j: (j, i, 0)),
        out_shape=jax.ShapeDtypeStruct((b, s, d), x.dtype),
    )(x, pos_table)


NW = 32
CH = 16


def _sc_add_pe(x, pos_table):
    b, s, d = x.shape
    rows = b * s
    rw = rows // NW
    nch = rw // CH

    x2 = x.reshape(rows, d)
    mesh = plsc.VectorSubcoreMesh(core_axis_name="c", subcore_axis_name="s")

    nb = b
    sw = s // NW
    nsc = sw // CH

    @functools.partial(
        pl.kernel,
        out_type=jax.ShapeDtypeStruct((rows, d), jnp.float32),
        mesh=mesh,
        scratch_types=[
            pltpu.VMEM((4, CH, d), jnp.float32),
            pltpu.VMEM((2, CH, d), jnp.float32),
            pltpu.SemaphoreType.DMA,
            pltpu.SemaphoreType.DMA,
            pltpu.SemaphoreType.DMA,
            pltpu.SemaphoreType.DMA,
            pltpu.SemaphoreType.DMA,
            pltpu.SemaphoreType.DMA,
            pltpu.SemaphoreType.DMA,
            pltpu.SemaphoreType.DMA,
            pltpu.SemaphoreType.DMA,
            pltpu.SemaphoreType.DMA,
        ],
    )
    def k(x_hbm, tab_hbm, out_hbm, xbuf, tbuf,
          lx0, lx1, lx2, lx3, lt0, lt1, st0, st1, st2, st3):
        wid = lax.axis_index("s") * 2 + lax.axis_index("c")
        s0 = wid * sw
        lxs = (lx0, lx1, lx2, lx3)
        lts = (lt0, lt1)
        sts = (st0, st1, st2, st3)

        def xrow(c):
            return (c % nb) * s + s0 + (c // nb) * CH

        def issue_x(c, slot):
            pltpu.async_copy(x_hbm.at[pl.ds(xrow(c), CH)],
                             xbuf.at[slot], lxs[slot])

        def issue_t(sc, tslot):
            pltpu.async_copy(tab_hbm.at[pl.ds(s0 + sc * CH, CH)],
                             tbuf.at[tslot], lts[tslot])

        issue_x(0, 0)
        issue_x(1, 1)
        issue_t(0, 0)

        nj = d // 16

        @pl.loop(0, nsc // 2)
        def _scpair(i2):
            for scp in (0, 1):
                sc = i2 * 2 + scp
                tslot = scp
                for bb in range(nb):
                    c = sc * nb + bb
                    slot = bb

                    @pl.when(c >= 2)
                    def _():
                        pltpu.make_async_copy(
                            xbuf.at[(bb + 2) % 4],
                            out_hbm.at[pl.ds(xrow(c - 2), CH)],
                            sts[(bb + 2) % 4],
                        ).wait()

                    @pl.when(c + 2 < nch)
                    def _():
                        issue_x(c + 2, (bb + 2) % 4)

                    if bb == nb - 1:
                        @pl.when(sc + 1 < nsc)
                        def _():
                            issue_t(sc + 1, 1 - tslot)

                    pltpu.make_async_copy(
                        x_hbm.at[pl.ds(xrow(c), CH)],
                        xbuf.at[slot], lxs[slot]).wait()

                    if bb == 0:
                        pltpu.make_async_copy(
                            tab_hbm.at[pl.ds(s0 + sc * CH, CH)],
                            tbuf.at[tslot], lts[tslot]).wait()

                    @plsc.parallel_loop(0, CH * nj, unroll=8)
                    def _q(q):
                        r = q // nj
                        j = (q % nj) * 16
                        plsc.addupdate(
                            xbuf.at[slot, r, pl.ds(j, 16)],
                            tbuf[tslot, r, pl.ds(j, 16)],
                        )

                    pltpu.async_copy(
                        xbuf.at[slot],
                        out_hbm.at[pl.ds(xrow(c), CH)],
                        sts[slot])

        for c in (nch - 2, nch - 1):
            pltpu.make_async_copy(
                xbuf.at[c % 4],
                out_hbm.at[pl.ds(xrow(c), CH)],
                sts[c % 4],
            ).wait()

    return k(x2, pos_table).reshape(b, s, d)


def kernel(x, pos_table):
    return _sc_add_pe(x, pos_table)
